# Initial kernel scaffold; baseline (speedup 1.0000x reference)
import functools

import jax
import jax.numpy as jnp
from jax import lax
from jax.experimental import pallas as pl
from jax.experimental.pallas import tpu as pltpu
from jax.experimental.pallas import tpu_sc as plsc

T = 2048
D = 768
E = 64
K = 2
HID = 1536
TK = T * K
BLK = 64
NPAD = 8192
NB = NPAD // BLK
CH = 256
NCH = T // CH

NC, NS, L = 2, 16, 16
NW = NC * NS
SPW = TK // NW
TPW = T // NW


def _router_kernel(x_ref, gw_ref, bias_ref,
                   s0_ref, s1_ref, p0_ref, p1_ref, be_ref, tot_ref):
    x = x_ref[...]
    logits = lax.dot_general(x, gw_ref[...], (((1,), (1,)), ((), ())),
                             preferred_element_type=jnp.float32)
    scores = jax.nn.sigmoid(logits)
    biased = scores + bias_ref[...]
    col = lax.broadcasted_iota(jnp.float32, (T, E), 1)
    m0 = biased >= jnp.max(biased, axis=1, keepdims=True)
    e0 = jnp.min(jnp.where(m0, col, float(E)), axis=1, keepdims=True)
    oh0 = col == e0
    s0 = jnp.sum(jnp.where(oh0, scores, 0.0), axis=1, keepdims=True)
    biased1 = jnp.where(oh0, -jnp.inf, biased)
    m1 = biased1 >= jnp.max(biased1, axis=1, keepdims=True)
    e1 = jnp.min(jnp.where(m1, col, float(E)), axis=1, keepdims=True)
    oh1 = col == e1
    s1 = jnp.sum(jnp.where(oh1, scores, 0.0), axis=1, keepdims=True)

    hh = oh0.astype(jnp.float32) + oh1.astype(jnp.float32)
    ri = lax.broadcasted_iota(jnp.int32, (CH, CH), 0)
    ci = lax.broadcasted_iota(jnp.int32, (CH, CH), 1)
    ltri = (ri > ci).astype(jnp.float32)
    base = jnp.zeros((1, E), jnp.float32)
    r0_parts, r1_parts = [], []
    for c in range(NCH):
        hc = hh[c * CH:(c + 1) * CH, :]
        cumc = lax.dot_general(ltri, hc, (((1,), (0,)), ((), ())),
                               preferred_element_type=jnp.float32) + base
        oh0c = oh0[c * CH:(c + 1) * CH, :]
        oh1c = oh1[c * CH:(c + 1) * CH, :]
        r0_parts.append(jnp.sum(jnp.where(oh0c, cumc, 0.0), axis=1, keepdims=True))
        r1_parts.append(jnp.sum(jnp.where(oh1c, cumc, 0.0), axis=1, keepdims=True))
        base = base + jnp.sum(hc, axis=0, keepdims=True)
    counts = base
    rank0 = jnp.concatenate(r0_parts, axis=0)
    rank1 = jnp.concatenate(r1_parts, axis=0)

    pc = jnp.floor((counts + (BLK - 1)) / BLK) * BLK
    er = lax.broadcasted_iota(jnp.int32, (E, E), 0)
    ec = lax.broadcasted_iota(jnp.int32, (E, E), 1)
    ustr = (er < ec).astype(jnp.float32)
    offs = lax.dot_general(pc, ustr, (((1,), (0,)), ((), ())),
                           preferred_element_type=jnp.float32)
    tot = jnp.sum(pc, axis=1, keepdims=True)
    pos0 = jnp.sum(jnp.where(oh0, offs, 0.0), axis=1, keepdims=True) + rank0
    pos1 = jnp.sum(jnp.where(oh1, offs, 0.0), axis=1, keepdims=True) + rank1
    bb = lax.broadcasted_iota(jnp.float32, (NB, 1), 0) * float(BLK)
    be = jnp.sum((offs <= bb).astype(jnp.float32), axis=1, keepdims=True) - 1.0

    s0_ref[...] = s0
    s1_ref[...] = s1
    p0_ref[...] = pos0.astype(jnp.int32)
    p1_ref[...] = pos1.astype(jnp.int32)
    be_ref[...] = be.astype(jnp.int32)
    tot_ref[...] = tot.astype(jnp.int32)


_router = pl.pallas_call(
    _router_kernel,
    out_shape=(
        jax.ShapeDtypeStruct((T, 1), jnp.float32),
        jax.ShapeDtypeStruct((T, 1), jnp.float32),
        jax.ShapeDtypeStruct((T, 1), jnp.int32),
        jax.ShapeDtypeStruct((T, 1), jnp.int32),
        jax.ShapeDtypeStruct((NB, 1), jnp.int32),
        jax.ShapeDtypeStruct((1, 1), jnp.int32),
    ),
)


def _mm_kernel(be_sm, tot_sm, rows_ref, wu_ref, wd_ref, out_ref):
    b = pl.program_id(0)

    @pl.when(b * BLK < tot_sm[0])
    def _():
        rows = rows_ref[...]
        h = lax.dot_general(rows, wu_ref[0], (((1,), (1,)), ((), ())),
                            preferred_element_type=jnp.float32)
        h = jnp.square(jnp.maximum(h, 0.0))
        out_ref[...] = lax.dot_general(h, wd_ref[0], (((1,), (1,)), ((), ())),
                                       preferred_element_type=jnp.float32)


_grouped_mm = pl.pallas_call(
    _mm_kernel,
    grid_spec=pltpu.PrefetchScalarGridSpec(
        num_scalar_prefetch=2,
        grid=(NB,),
        in_specs=[
            pl.BlockSpec((BLK, D), lambda b, be, tot: (b, 0)),
            pl.BlockSpec((1, HID, D), lambda b, be, tot: (be[b], 0, 0)),
            pl.BlockSpec((1, D, HID), lambda b, be, tot: (be[b], 0, 0)),
        ],
        out_specs=pl.BlockSpec((BLK, D), lambda b, be, tot: (b, 0)),
    ),
    out_shape=jax.ShapeDtypeStruct((NPAD, D), jnp.float32),
)


_sc_mesh = plsc.VectorSubcoreMesh(core_axis_name="c", subcore_axis_name="s")


@functools.partial(
    pl.kernel,
    mesh=_sc_mesh,
    out_type=jax.ShapeDtypeStruct((NPAD, D), jnp.float32),
    scratch_types=[
        pltpu.VMEM((SPW,), jnp.int32),
        pltpu.VMEM((SPW,), jnp.int32),
        pltpu.VMEM((SPW,), jnp.float32),
        pltpu.VMEM((SPW, D), jnp.float32),
        pltpu.SemaphoreType.DMA,
    ],
)
def _dispatch(x_hbm, tid_hbm, pos_hbm, sc_hbm, out_hbm,
              tid_v, pos_v, sc_v, rows_v, sem):
    wid = lax.axis_index("s") * NC + lax.axis_index("c")
    base = wid * SPW
    pltpu.sync_copy(tid_hbm.at[pl.ds(base, SPW)], tid_v)
    pltpu.sync_copy(pos_hbm.at[pl.ds(base, SPW)], pos_v)
    pltpu.sync_copy(sc_hbm.at[pl.ds(base, SPW)], sc_v)
    pltpu.async_copy(x_hbm.at[tid_v], rows_v, sem).wait()

    def scale_row(r, carry):
        s = plsc.load_gather(sc_v, [jnp.zeros((L,), jnp.int32) + r])
        for cc in range(D // L):
            sl = pl.ds(cc * L, L)
            rows_v[r, sl] = rows_v[r, sl] * s
        return carry

    lax.fori_loop(0, SPW, scale_row, 0)
    pltpu.async_copy(rows_v, out_hbm.at[pos_v], sem).wait()


@functools.partial(
    pl.kernel,
    mesh=_sc_mesh,
    out_type=jax.ShapeDtypeStruct((T, D), jnp.float32),
    scratch_types=[
        pltpu.VMEM((TPW,), jnp.int32),
        pltpu.VMEM((TPW,), jnp.int32),
        pltpu.VMEM((TPW, D), jnp.float32),
        pltpu.VMEM((TPW, D), jnp.float32),
        pltpu.SemaphoreType.DMA,
    ],
)
def _combine(ro_hbm, p0_hbm, p1_hbm, out_hbm, p0_v, p1_v, a_v, b_v, sem):
    wid = lax.axis_index("s") * NC + lax.axis_index("c")
    base = wid * TPW
    pltpu.sync_copy(p0_hbm.at[pl.ds(base, TPW)], p0_v)
    pltpu.sync_copy(p1_hbm.at[pl.ds(base, TPW)], p1_v)
    pltpu.async_copy(ro_hbm.at[p0_v], a_v, sem).wait()
    pltpu.async_copy(ro_hbm.at[p1_v], b_v, sem).wait()

    def add_row(r, carry):
        for cc in range(D // L):
            sl = pl.ds(cc * L, L)
            a_v[r, sl] = a_v[r, sl] + b_v[r, sl]
        return carry

    lax.fori_loop(0, TPW, add_row, 0)
    pltpu.sync_copy(a_v, out_hbm.at[pl.ds(base, TPW)])


def kernel(x, gate_w, w_up, w_down, expert_bias):
    x_flat = x.reshape(T, D)
    s0, s1, p0, p1, be, tot = _router(x_flat, gate_w, expert_bias.reshape(1, E))
    sc_flat = jnp.concatenate([s0, s1], axis=1).reshape(TK)
    pos_flat = jnp.concatenate([p0, p1], axis=1).reshape(TK)
    tid = jnp.arange(TK, dtype=jnp.int32) // K
    routed = _dispatch(x_flat, tid, pos_flat, sc_flat)
    ro2 = _grouped_mm(be.reshape(NB), tot.reshape(1), routed, w_up, w_down)
    y = _combine(ro2, p0.reshape(T), p1.reshape(T))
    return y.reshape(1, T, D)


# speedup vs baseline: 6.2555x; 6.2555x over previous
import functools

import jax
import jax.numpy as jnp
from jax import lax
from jax.experimental import pallas as pl
from jax.experimental.pallas import tpu as pltpu
from jax.experimental.pallas import tpu_sc as plsc

T = 2048
D = 768
E = 64
K = 2
HID = 1536
TK = T * K
BLK = 64
NPAD = 8192
NB = NPAD // BLK
CH = 256
NCH = T // CH

NC, NS, L = 2, 16, 16
NW = NC * NS
SPW = TK // NW
TPW = T // NW


def _router_kernel(x_ref, gw_ref, bias_ref,
                   s0_ref, s1_ref, p0_ref, p1_ref, be_ref, tot_ref):
    x = x_ref[...]
    logits = lax.dot_general(x, gw_ref[...], (((1,), (1,)), ((), ())),
                             preferred_element_type=jnp.float32)
    scores = jax.nn.sigmoid(logits)
    biased = scores + bias_ref[...]
    col = lax.broadcasted_iota(jnp.int32, (T, E), 1).astype(jnp.float32)
    m0 = biased >= jnp.max(biased, axis=1, keepdims=True)
    e0 = jnp.min(jnp.where(m0, col, float(E)), axis=1, keepdims=True)
    oh0 = col == e0
    s0 = jnp.sum(jnp.where(oh0, scores, 0.0), axis=1, keepdims=True)
    biased1 = jnp.where(oh0, -jnp.inf, biased)
    m1 = biased1 >= jnp.max(biased1, axis=1, keepdims=True)
    e1 = jnp.min(jnp.where(m1, col, float(E)), axis=1, keepdims=True)
    oh1 = col == e1
    s1 = jnp.sum(jnp.where(oh1, scores, 0.0), axis=1, keepdims=True)

    hh = oh0.astype(jnp.float32) + oh1.astype(jnp.float32)
    ri = lax.broadcasted_iota(jnp.int32, (CH, CH), 0)
    ci = lax.broadcasted_iota(jnp.int32, (CH, CH), 1)
    ltri = (ri > ci).astype(jnp.float32)
    base = jnp.zeros((1, E), jnp.float32)
    r0_parts, r1_parts = [], []
    for c in range(NCH):
        hc = hh[c * CH:(c + 1) * CH, :]
        cumc = lax.dot_general(ltri, hc, (((1,), (0,)), ((), ())),
                               preferred_element_type=jnp.float32) + base
        oh0c = oh0[c * CH:(c + 1) * CH, :]
        oh1c = oh1[c * CH:(c + 1) * CH, :]
        r0_parts.append(jnp.sum(jnp.where(oh0c, cumc, 0.0), axis=1, keepdims=True))
        r1_parts.append(jnp.sum(jnp.where(oh1c, cumc, 0.0), axis=1, keepdims=True))
        base = base + jnp.sum(hc, axis=0, keepdims=True)
    counts = base
    rank0 = jnp.concatenate(r0_parts, axis=0)
    rank1 = jnp.concatenate(r1_parts, axis=0)

    pc = jnp.floor((counts + (BLK - 1)) / BLK) * BLK
    er = lax.broadcasted_iota(jnp.int32, (E, E), 0)
    ec = lax.broadcasted_iota(jnp.int32, (E, E), 1)
    ustr = (er < ec).astype(jnp.float32)
    offs = lax.dot_general(pc, ustr, (((1,), (0,)), ((), ())),
                           preferred_element_type=jnp.float32)
    tot = jnp.sum(pc, axis=1, keepdims=True)
    pos0 = jnp.sum(jnp.where(oh0, offs, 0.0), axis=1, keepdims=True) + rank0
    pos1 = jnp.sum(jnp.where(oh1, offs, 0.0), axis=1, keepdims=True) + rank1
    bb = lax.broadcasted_iota(jnp.int32, (NB, 1), 0).astype(jnp.float32) * float(BLK)
    be = jnp.sum((offs <= bb).astype(jnp.float32), axis=1, keepdims=True) - 1.0

    s0_ref[...] = jnp.broadcast_to(s0, (T, L))
    s1_ref[...] = jnp.broadcast_to(s1, (T, L))
    p0_ref[...] = pos0.astype(jnp.int32)
    p1_ref[...] = pos1.astype(jnp.int32)
    be_ref[...] = be.astype(jnp.int32)
    tot_ref[...] = tot.astype(jnp.int32)


_router = pl.pallas_call(
    _router_kernel,
    out_shape=(
        jax.ShapeDtypeStruct((T, L), jnp.float32),
        jax.ShapeDtypeStruct((T, L), jnp.float32),
        jax.ShapeDtypeStruct((T, 1), jnp.int32),
        jax.ShapeDtypeStruct((T, 1), jnp.int32),
        jax.ShapeDtypeStruct((NB, 1), jnp.int32),
        jax.ShapeDtypeStruct((1, 1), jnp.int32),
    ),
)


def _mm_kernel(be_sm, tot_sm, rows_ref, wu_ref, wd_ref, out_ref):
    b = pl.program_id(0)

    @pl.when(b * BLK < tot_sm[0])
    def _():
        rows = rows_ref[...]
        h = lax.dot_general(rows, wu_ref[0], (((1,), (1,)), ((), ())),
                            preferred_element_type=jnp.float32)
        h = jnp.square(jnp.maximum(h, 0.0))
        out_ref[...] = lax.dot_general(h, wd_ref[0], (((1,), (1,)), ((), ())),
                                       preferred_element_type=jnp.float32)


_grouped_mm = pl.pallas_call(
    _mm_kernel,
    grid_spec=pltpu.PrefetchScalarGridSpec(
        num_scalar_prefetch=2,
        grid=(NB,),
        in_specs=[
            pl.BlockSpec((BLK, D), lambda b, be, tot: (b, 0)),
            pl.BlockSpec((1, HID, D), lambda b, be, tot: (be[b], 0, 0)),
            pl.BlockSpec((1, D, HID), lambda b, be, tot: (be[b], 0, 0)),
        ],
        out_specs=pl.BlockSpec((BLK, D), lambda b, be, tot: (b, 0)),
    ),
    out_shape=jax.ShapeDtypeStruct((NPAD, D), jnp.float32),
)


@functools.cache
def _build_dispatch():
    mesh = plsc.VectorSubcoreMesh(core_axis_name="c", subcore_axis_name="s")

    @functools.partial(
        pl.kernel,
        mesh=mesh,
        out_type=jax.ShapeDtypeStruct((NPAD, D), jnp.float32),
        scratch_types=[
            pltpu.VMEM((SPW,), jnp.int32),
            pltpu.VMEM((SPW,), jnp.int32),
            pltpu.VMEM((SPW, L), jnp.float32),
            pltpu.VMEM((SPW, D), jnp.float32),
            pltpu.SemaphoreType.DMA,
        ],
    )
    def _dispatch(x_hbm, tid_hbm, pos_hbm, sc_hbm, out_hbm,
                  tid_v, pos_v, sc_v, rows_v, sem):
        wid = lax.axis_index("s") * NC + lax.axis_index("c")
        base = wid * SPW
        pltpu.sync_copy(tid_hbm.at[pl.ds(base, SPW)], tid_v)
        pltpu.sync_copy(pos_hbm.at[pl.ds(base, SPW)], pos_v)
        pltpu.sync_copy(sc_hbm.at[pl.ds(base, SPW)], sc_v)
        pltpu.async_copy(x_hbm.at[tid_v], rows_v, sem).wait()

        def scale_row(r, carry):
            s = sc_v[r, :]
            for cc in range(D // L):
                sl = pl.ds(cc * L, L)
                rows_v[r, sl] = rows_v[r, sl] * s
            return carry

        lax.fori_loop(0, SPW, scale_row, 0)
        pltpu.async_copy(rows_v, out_hbm.at[pos_v], sem).wait()

    return _dispatch


@functools.cache
def _build_combine():
    mesh = plsc.VectorSubcoreMesh(core_axis_name="c", subcore_axis_name="s")

    @functools.partial(
        pl.kernel,
        mesh=mesh,
        out_type=jax.ShapeDtypeStruct((T, D), jnp.float32),
        scratch_types=[
            pltpu.VMEM((TPW,), jnp.int32),
            pltpu.VMEM((TPW,), jnp.int32),
            pltpu.VMEM((TPW, D), jnp.float32),
            pltpu.VMEM((TPW, D), jnp.float32),
            pltpu.SemaphoreType.DMA,
        ],
    )
    def _combine(ro_hbm, p0_hbm, p1_hbm, out_hbm, p0_v, p1_v, a_v, b_v, sem):
        wid = lax.axis_index("s") * NC + lax.axis_index("c")
        base = wid * TPW
        pltpu.sync_copy(p0_hbm.at[pl.ds(base, TPW)], p0_v)
        pltpu.sync_copy(p1_hbm.at[pl.ds(base, TPW)], p1_v)
        pltpu.async_copy(ro_hbm.at[p0_v], a_v, sem).wait()
        pltpu.async_copy(ro_hbm.at[p1_v], b_v, sem).wait()

        def add_row(r, carry):
            for cc in range(D // L):
                sl = pl.ds(cc * L, L)
                a_v[r, sl] = a_v[r, sl] + b_v[r, sl]
            return carry

        lax.fori_loop(0, TPW, add_row, 0)
        pltpu.sync_copy(a_v, out_hbm.at[pl.ds(base, TPW)])

    return _combine


def kernel(x, gate_w, w_up, w_down, expert_bias):
    x_flat = x.reshape(T, D)
    s0, s1, p0, p1, be, tot = _router(x_flat, gate_w, expert_bias.reshape(1, E))
    sc_rep = jnp.stack([s0, s1], axis=1).reshape(TK, L)
    pos_flat = jnp.concatenate([p0, p1], axis=1).reshape(TK)
    tid = jnp.arange(TK, dtype=jnp.int32) // K
    routed = _build_dispatch()(x_flat, tid, pos_flat, sc_rep)
    ro2 = _grouped_mm(be.reshape(NB), tot.reshape(1), routed, w_up, w_down)
    y = _build_combine()(ro2, p0.reshape(T), p1.reshape(T))
    return y.reshape(1, T, D)
